# Initial kernel scaffold; baseline (speedup 1.0000x reference)
import jax
import jax.numpy as jnp
from jax.experimental import pallas as pl


def kernel(input_ids, attention_mask, init_workspace, word_table):
    raise NotImplementedError("write your pallas kernel here")


# speedup vs baseline: 1.3621x; 1.3621x over previous
import functools
import jax
import jax.numpy as jnp
from jax import lax
from jax.experimental import pallas as pl
from jax.experimental.pallas import tpu as pltpu
from jax.experimental.pallas import tpu_sc as plsc

B, S = 4, 4096
WS, WH = 64, 128
V, TH = 1000000, 128

NC, NS = 2, 16
NW = NC * NS
N = B * S
BPW = N // NW
CH = 128
NCH = BPW // CH


def _body(ids_hbm, ws_hbm, table_hbm, ws_out, emb_out, idx_v, rows_v, ws_v,
          gsem, osem):
    wid = lax.axis_index("s") * NC + lax.axis_index("c")

    pltpu.sync_copy(ids_hbm.at[wid], idx_v)

    copies = []
    for j in range(NCH):
        copies.append(
            pltpu.async_copy(
                table_hbm.at[idx_v.at[j]],
                rows_v.at[pl.ds(j * CH, CH)],
                gsem,
            )
        )
    for c in copies:
        c.wait()

    out_copy = pltpu.async_copy(rows_v, emb_out.at[wid], osem)

    @pl.when(wid < B)
    def _():
        pltpu.sync_copy(ws_hbm, ws_v)
        pltpu.sync_copy(ws_v, ws_out.at[wid])

    out_copy.wait()


@jax.jit
def _run(ids, ws_flat, table):
    kern = pl.kernel(
        _body,
        out_type=(
            jax.ShapeDtypeStruct((B, WS * WH), jnp.float32),
            jax.ShapeDtypeStruct((NW, BPW, TH), jnp.float32),
        ),
        mesh=plsc.VectorSubcoreMesh(core_axis_name="c", subcore_axis_name="s"),
        scratch_types=[
            pltpu.VMEM((NCH, CH), jnp.int32),
            pltpu.VMEM((BPW, TH), jnp.float32),
            pltpu.VMEM((WS * WH,), jnp.float32),
            pltpu.SemaphoreType.DMA,
            pltpu.SemaphoreType.DMA,
        ],
    )
    return kern(ids, ws_flat, table)


def kernel(input_ids, attention_mask, init_workspace, word_table):
    ids = input_ids.reshape(NW, NCH, CH)
    ws_flat = init_workspace.reshape(WS * WH)
    ws_out, emb = _run(ids, ws_flat, word_table)
    workspace = ws_out.reshape(B, WS, WH)
    embeddings = emb.reshape(B, S, TH)
    return (workspace, embeddings)


# speedup vs baseline: 1.3953x; 1.0244x over previous
import functools
import jax
import jax.numpy as jnp
from jax import lax
from jax.experimental import pallas as pl
from jax.experimental.pallas import tpu as pltpu
from jax.experimental.pallas import tpu_sc as plsc

B, S = 4, 4096
WS, WH = 64, 128
V, TH = 1000000, 128

NC, NS = 2, 16
NW = NC * NS
N = B * S
BPW = N // NW
CH = 128
NCH = BPW // CH
WPIECE = WS * WH // (NW // B)


def _body(ids_hbm, ws_hbm, table_hbm, ws_out, emb_out, idx_v, rows_v, ws_v,
          g0, g1, g2, g3, osem):
    gsems = [g0, g1, g2, g3]
    wid = lax.axis_index("s") * NC + lax.axis_index("c")

    pltpu.sync_copy(ids_hbm.at[wid], idx_v)

    gathers = []
    for j in range(NCH):
        gathers.append(
            pltpu.async_copy(
                table_hbm.at[idx_v.at[j]],
                rows_v.at[pl.ds(j * CH, CH)],
                gsems[j],
            )
        )

    piece = lax.rem(wid, NW // B) * WPIECE
    batch = lax.div(wid, NW // B)
    pltpu.sync_copy(ws_hbm.at[pl.ds(piece, WPIECE)], ws_v)
    pltpu.sync_copy(ws_v, ws_out.at[batch, pl.ds(piece, WPIECE)])

    out_copies = []
    for j in range(NCH):
        gathers[j].wait()
        out_copies.append(
            pltpu.async_copy(
                rows_v.at[pl.ds(j * CH, CH)],
                emb_out.at[wid, pl.ds(j * CH, CH)],
                osem,
            )
        )
    for c in out_copies:
        c.wait()


@jax.jit
def _run(ids, ws_flat, table):
    kern = pl.kernel(
        _body,
        out_type=(
            jax.ShapeDtypeStruct((B, WS * WH), jnp.float32),
            jax.ShapeDtypeStruct((NW, BPW, TH), jnp.float32),
        ),
        mesh=plsc.VectorSubcoreMesh(core_axis_name="c", subcore_axis_name="s"),
        scratch_types=[
            pltpu.VMEM((NCH, CH), jnp.int32),
            pltpu.VMEM((BPW, TH), jnp.float32),
            pltpu.VMEM((WPIECE,), jnp.float32),
            pltpu.SemaphoreType.DMA,
            pltpu.SemaphoreType.DMA,
            pltpu.SemaphoreType.DMA,
            pltpu.SemaphoreType.DMA,
            pltpu.SemaphoreType.DMA,
        ],
    )
    return kern(ids, ws_flat, table)


def kernel(input_ids, attention_mask, init_workspace, word_table):
    ids = input_ids.reshape(NW, NCH, CH)
    ws_flat = init_workspace.reshape(WS * WH)
    ws_out, emb = _run(ids, ws_flat, word_table)
    workspace = ws_out.reshape(B, WS, WH)
    embeddings = emb.reshape(B, S, TH)
    return (workspace, embeddings)
